# Initial kernel scaffold; baseline (speedup 1.0000x reference)
import jax
import jax.numpy as jnp
from jax.experimental import pallas as pl


def kernel(y_pred, y_true):
    raise NotImplementedError("write your pallas kernel here")


# speedup vs baseline: 4.4795x; 4.4795x over previous
import jax
import jax.numpy as jnp
from jax.experimental import pallas as pl


def _ce_body(y_pred_ref, y_true_ref, out_ref):
    b = pl.program_id(0)
    h = pl.program_id(1)
    x = y_pred_ref[0]
    y = y_true_ref[0]
    m = jnp.max(x, axis=0)
    s = jnp.sum(jnp.exp(x - m[None]), axis=0)
    lse = m + jnp.log(s)
    cls = jax.lax.broadcasted_iota(jnp.int32, x.shape, 0)
    sel = jnp.sum(jnp.where(cls == y[None], x, 0.0), axis=0)
    partial = jnp.sum(lse - sel, axis=0, keepdims=True)

    @pl.when((b == 0) & (h == 0))
    def _():
        out_ref[...] = jnp.zeros_like(out_ref)

    out_ref[...] += partial


def kernel(y_pred, y_true):
    B, C, H, W = y_pred.shape
    Hb = 128
    out = pl.pallas_call(
        _ce_body,
        grid=(B, H // Hb),
        in_specs=[
            pl.BlockSpec((1, C, Hb, W), lambda b, h: (b, 0, h, 0)),
            pl.BlockSpec((1, Hb, W), lambda b, h: (b, h, 0)),
        ],
        out_specs=pl.BlockSpec((1, W), lambda b, h: (0, 0)),
        out_shape=jax.ShapeDtypeStruct((1, W), jnp.float32),
    )(y_pred, y_true)
    return jnp.sum(out) / (B * H * W)


# speedup vs baseline: 5.3027x; 1.1838x over previous
import jax
import jax.numpy as jnp
from jax.experimental import pallas as pl


def _ce_body(y_pred_ref, y_true_ref, out_ref):
    b = pl.program_id(0)
    h = pl.program_id(1)
    C, Hb, W = y_pred_ref.shape[1:]
    P = 8

    partial = jnp.zeros((1, W), jnp.float32)
    for p in range(Hb // P):
        rows = pl.ds(p * P, P)
        y = y_true_ref[0, rows, :]
        m = jnp.full((P, W), -jnp.inf, jnp.float32)
        sel = jnp.zeros((P, W), jnp.float32)
        for c in range(C):
            xc = y_pred_ref[0, c, rows, :]
            m = jnp.maximum(m, xc)
            sel += jnp.where(y == c, xc, 0.0)
        s = jnp.zeros((P, W), jnp.float32)
        for c in range(C):
            xc = y_pred_ref[0, c, rows, :]
            s += jnp.exp(xc - m)
        partial += jnp.sum(m + jnp.log(s) - sel, axis=0, keepdims=True)

    @pl.when((b == 0) & (h == 0))
    def _():
        out_ref[...] = jnp.zeros_like(out_ref)

    out_ref[...] += partial


def kernel(y_pred, y_true):
    B, C, H, W = y_pred.shape
    Hb = 128
    out = pl.pallas_call(
        _ce_body,
        grid=(B, H // Hb),
        in_specs=[
            pl.BlockSpec((1, C, Hb, W), lambda b, h: (b, 0, h, 0)),
            pl.BlockSpec((1, Hb, W), lambda b, h: (b, h, 0)),
        ],
        out_specs=pl.BlockSpec((1, W), lambda b, h: (0, 0)),
        out_shape=jax.ShapeDtypeStruct((1, W), jnp.float32),
    )(y_pred, y_true)
    return jnp.sum(out) / (B * H * W)
